# Initial kernel scaffold; baseline (speedup 1.0000x reference)
import jax
import jax.numpy as jnp
from jax.experimental import pallas as pl


def kernel(inputs, table_2, table_3, table_4):
    raise NotImplementedError("write your pallas kernel here")


# speedup vs baseline: 1.4224x; 1.4224x over previous
import functools

import jax
import jax.numpy as jnp
from jax import lax
from jax.experimental import pallas as pl
from jax.experimental.pallas import tpu as pltpu
from jax.experimental.pallas import tpu_sc as plsc

_VOCAB = 1000000
_D = 16
_B, _S = 4096, 200
_NTOK = _B * _S
_BLK = 128
_NROWS = _NTOK // _BLK
_P2 = 257 * 257
_P3 = (257 ** 3) % _VOCAB

_info = plsc.get_sparse_core_info()
_NC, _NS = _info.num_cores, _info.num_subcores
_NW = _NC * _NS
_ROWS_PER_W = _NROWS // _NW
_CHUNK_ROWS = 40
_NCHUNK = _ROWS_PER_W // _CHUNK_ROWS


def _hash_body(x0, x1, x2, x3, h2o, h3o, h4o):
    h2 = x1[...] * 257 + x0[...]
    h3 = (h2 + x2[...] * _P2) % _VOCAB
    h4 = (h3 + x3[...] * _P3) % _VOCAB
    h2o[...] = h2
    h3o[...] = h3
    h4o[...] = h4


def _hash_tc(x0, x1, x2, x3):
    grid = 8
    rows = _B // grid
    spec = pl.BlockSpec((rows, _S), lambda i: (i, jnp.int32(0)))
    return pl.pallas_call(
        _hash_body,
        grid=(grid,),
        in_specs=[spec] * 4,
        out_specs=[spec] * 3,
        out_shape=[jax.ShapeDtypeStruct((_B, _S), jnp.int32)] * 3,
    )(x0, x1, x2, x3)


@functools.partial(
    pl.kernel,
    out_type=jax.ShapeDtypeStruct((_NROWS, _BLK, _D), jnp.float32),
    mesh=plsc.VectorSubcoreMesh(core_axis_name="c", subcore_axis_name="s"),
    compiler_params=pltpu.CompilerParams(use_tc_tiling_on_sc=False),
    scratch_types=[
        pltpu.VMEM((_CHUNK_ROWS, _BLK), jnp.int32),
        pltpu.VMEM((_CHUNK_ROWS, _BLK), jnp.int32),
        pltpu.VMEM((_CHUNK_ROWS, _BLK), jnp.int32),
        pltpu.VMEM((_BLK, _D), jnp.float32),
        pltpu.VMEM((_BLK, _D), jnp.float32),
        pltpu.VMEM((_BLK, _D), jnp.float32),
        pltpu.SemaphoreType.DMA,
        pltpu.SemaphoreType.DMA,
        pltpu.SemaphoreType.DMA,
    ],
)
def _sc_embed(h2_h, h3_h, h4_h, t2_h, t3_h, t4_h, out_h,
              vi2, vi3, vi4, r2, r3, r4, s2, s3, s4):
    i32 = jnp.int32
    wid = lax.axis_index("s") * i32(_NC) + lax.axis_index("c")
    row0 = wid * i32(_ROWS_PER_W)

    def chunk_body(c, carry):
        rbase = pl.multiple_of(row0 + c * i32(_CHUNK_ROWS), 8)
        pltpu.sync_copy(h2_h.at[pl.ds(rbase, _CHUNK_ROWS)], vi2)
        pltpu.sync_copy(h3_h.at[pl.ds(rbase, _CHUNK_ROWS)], vi3)
        pltpu.sync_copy(h4_h.at[pl.ds(rbase, _CHUNK_ROWS)], vi4)

        def blk_body(b, carry2):
            c2 = pltpu.async_copy(t2_h.at[vi2.at[b]], r2, s2)
            c3 = pltpu.async_copy(t3_h.at[vi3.at[b]], r3, s3)
            c4 = pltpu.async_copy(t4_h.at[vi4.at[b]], r4, s4)
            c2.wait()
            c3.wait()
            c4.wait()

            def acc_body(i, carry3):
                r2[i, :] = r2[i, :] + r3[i, :] + r4[i, :]
                return carry3

            lax.fori_loop(i32(0), i32(_BLK), acc_body, i32(0))
            pltpu.sync_copy(r2, out_h.at[rbase + b])
            return carry2

        lax.fori_loop(i32(0), i32(_CHUNK_ROWS), blk_body, i32(0))
        return carry

    lax.fori_loop(i32(0), i32(_NCHUNK), chunk_body, i32(0))


def kernel(inputs, table_2, table_3, table_4):
    x = inputs.astype(jnp.int32)
    x1 = jnp.pad(x, ((0, 0), (1, 0)))[:, :_S]
    x2 = jnp.pad(x, ((0, 0), (2, 0)))[:, :_S]
    x3 = jnp.pad(x, ((0, 0), (3, 0)))[:, :_S]
    h2, h3, h4 = _hash_tc(x, x1, x2, x3)
    out = _sc_embed(
        h2.reshape(_NROWS, _BLK),
        h3.reshape(_NROWS, _BLK),
        h4.reshape(_NROWS, _BLK),
        table_2, table_3, table_4,
    )
    return out.reshape(_B, _S, _D)


# speedup vs baseline: 1.5865x; 1.1154x over previous
import functools

import jax
import jax.numpy as jnp
from jax import lax
from jax.experimental import pallas as pl
from jax.experimental.pallas import tpu as pltpu
from jax.experimental.pallas import tpu_sc as plsc

_VOCAB = 1000000
_D = 16
_B, _S = 4096, 200
_NTOK = _B * _S
_BLK = 128
_NROWS = _NTOK // _BLK
_P2 = 257 * 257
_P3 = (257 ** 3) % _VOCAB

_info = plsc.get_sparse_core_info()
_NC, _NS = _info.num_cores, _info.num_subcores
_NW = _NC * _NS
_NB = _NROWS // _NW


def _hash_body(x0, x1, x2, x3, h2o, h3o, h4o):
    h2 = x1[...] * 257 + x0[...]
    h3 = (h2 + x2[...] * _P2) % _VOCAB
    h4 = (h3 + x3[...] * _P3) % _VOCAB
    h2o[...] = h2
    h3o[...] = h3
    h4o[...] = h4


def _hash_tc(x0, x1, x2, x3):
    grid = 8
    rows = _B // grid
    spec = pl.BlockSpec((rows, _S), lambda i: (i, jnp.int32(0)))
    return pl.pallas_call(
        _hash_body,
        grid=(grid,),
        in_specs=[spec] * 4,
        out_specs=[spec] * 3,
        out_shape=[jax.ShapeDtypeStruct((_B, _S), jnp.int32)] * 3,
    )(x0, x1, x2, x3)


@functools.partial(
    pl.kernel,
    out_type=jax.ShapeDtypeStruct((_NROWS, _BLK, _D), jnp.float32),
    mesh=plsc.VectorSubcoreMesh(core_axis_name="c", subcore_axis_name="s"),
    compiler_params=pltpu.CompilerParams(use_tc_tiling_on_sc=False),
    scratch_types=[
        pltpu.VMEM((_NB, _BLK), jnp.int32),
        pltpu.VMEM((_NB, _BLK), jnp.int32),
        pltpu.VMEM((_NB, _BLK), jnp.int32),
        pltpu.VMEM((_BLK, _D), jnp.float32),
        pltpu.VMEM((_BLK, _D), jnp.float32),
        pltpu.VMEM((_BLK, _D), jnp.float32),
        pltpu.VMEM((_BLK, _D), jnp.float32),
        pltpu.VMEM((_BLK, _D), jnp.float32),
        pltpu.VMEM((_BLK, _D), jnp.float32),
        pltpu.SemaphoreType.DMA,
        pltpu.SemaphoreType.DMA,
        pltpu.SemaphoreType.DMA,
        pltpu.SemaphoreType.DMA,
    ],
)
def _sc_embed(h2_h, h3_h, h4_h, t2_h, t3_h, t4_h, out_h,
              vi2, vi3, vi4,
              ra2, ra3, ra4, rb2, rb3, rb4,
              sga, sgb, soa, sob):
    i32 = jnp.int32
    wid = lax.axis_index("s") * i32(_NC) + lax.axis_index("c")
    row0 = wid * i32(_NB)

    pltpu.sync_copy(h2_h.at[pl.ds(row0, _NB)], vi2)
    pltpu.sync_copy(h3_h.at[pl.ds(row0, _NB)], vi3)
    pltpu.sync_copy(h4_h.at[pl.ds(row0, _NB)], vi4)

    def fire_gathers(b, r2, r3, r4, sg):
        pltpu.async_copy(t2_h.at[vi2.at[b]], r2, sg)
        pltpu.async_copy(t3_h.at[vi3.at[b]], r3, sg)
        pltpu.async_copy(t4_h.at[vi4.at[b]], r4, sg)

    def wait_gathers(b, r2, r3, r4, sg):
        pltpu.make_async_copy(t2_h.at[vi2.at[b]], r2, sg).wait()
        pltpu.make_async_copy(t3_h.at[vi3.at[b]], r3, sg).wait()
        pltpu.make_async_copy(t4_h.at[vi4.at[b]], r4, sg).wait()

    def step(b, r2, r3, r4, sg, q2, q3, q4, sq, so_other, so_mine):
        @pl.when(b >= i32(1))
        def _():
            pltpu.make_async_copy(q2, out_h.at[row0 + b - 1], so_other).wait()

        @pl.when(b + 1 < i32(_NB))
        def _():
            fire_gathers(b + 1, q2, q3, q4, sq)

        wait_gathers(b, r2, r3, r4, sg)

        def acc_body(i, carry):
            base = i * i32(8)
            for k in range(8):
                r2[base + k, :] = r2[base + k, :] + r3[base + k, :] + r4[base + k, :]
            return carry

        lax.fori_loop(i32(0), i32(_BLK // 8), acc_body, i32(0))
        pltpu.async_copy(r2, out_h.at[row0 + b], so_mine)

    fire_gathers(i32(0), ra2, ra3, ra4, sga)

    def outer(b2, carry):
        b = b2 * i32(2)
        step(b, ra2, ra3, ra4, sga, rb2, rb3, rb4, sgb, sob, soa)
        step(b + i32(1), rb2, rb3, rb4, sgb, ra2, ra3, ra4, sga, soa, sob)
        return carry

    lax.fori_loop(i32(0), i32(_NB // 2), outer, i32(0))
    pltpu.make_async_copy(rb2, out_h.at[row0 + i32(_NB - 1)], sob).wait()


def kernel(inputs, table_2, table_3, table_4):
    x = inputs.astype(jnp.int32)
    x1 = jnp.pad(x, ((0, 0), (1, 0)))[:, :_S]
    x2 = jnp.pad(x, ((0, 0), (2, 0)))[:, :_S]
    x3 = jnp.pad(x, ((0, 0), (3, 0)))[:, :_S]
    h2, h3, h4 = _hash_tc(x, x1, x2, x3)
    out = _sc_embed(
        h2.reshape(_NROWS, _BLK),
        h3.reshape(_NROWS, _BLK),
        h4.reshape(_NROWS, _BLK),
        table_2, table_3, table_4,
    )
    return out.reshape(_B, _S, _D)
